# baseline (speedup 1.0000x reference)
import functools

import jax
import jax.numpy as jnp
from jax.experimental import pallas as pl
from jax.experimental.pallas import tpu as pltpu

RANK = 4
ALPHA = 1.0
NSUB = 16
DIN = 1024
DFF = 4096
TS = 256


def _fused(sid_ref, x_ref, W1_ref, b1_ref, A1_ref, B1_ref,
           W2_ref, b2_ref, A2_ref, B2_ref, out_ref):
    x = x_ref[0]
    scale = ALPHA / RANK
    base = jnp.dot(x, W1_ref[...], preferred_element_type=jnp.float32)
    lo = jnp.dot(x, A1_ref[0], preferred_element_type=jnp.float32)
    lo = jnp.dot(lo.astype(jnp.bfloat16), B1_ref[0],
                 preferred_element_type=jnp.float32)
    h = base + b1_ref[...] + scale * lo
    h = 0.5 * h * (1.0 + jax.lax.erf(h * 0.7071067811865476))
    hb = h.astype(jnp.bfloat16)
    base2 = jnp.dot(hb, W2_ref[...], preferred_element_type=jnp.float32)
    lo2 = jnp.dot(hb, A2_ref[0], preferred_element_type=jnp.float32)
    lo2 = jnp.dot(lo2.astype(jnp.bfloat16), B2_ref[0],
                  preferred_element_type=jnp.float32)
    out_ref[0] = base2 + b2_ref[...] + scale * lo2


def kernel(x, subject_id, W1, b1, A1, B1, W2, b2, A2, B2):
    B, S, _ = x.shape
    bf = jnp.bfloat16
    xb = x.astype(bf)
    W1b, W2b = W1.astype(bf), W2.astype(bf)
    A1b, B1b = A1.astype(bf), B1.astype(bf)
    A2b, B2b = A2.astype(bf), B2.astype(bf)
    b1r = b1.reshape(1, DFF)
    b2r = b2.reshape(1, DIN)
    sid = subject_id.astype(jnp.int32)

    grid = (B, S // TS)
    grid_spec = pltpu.PrefetchScalarGridSpec(
        num_scalar_prefetch=1,
        grid=grid,
        in_specs=[
            pl.BlockSpec((1, TS, DIN), lambda b, t, sid: (b, t, 0)),
            pl.BlockSpec((DIN, DFF), lambda b, t, sid: (0, 0)),
            pl.BlockSpec((1, DFF), lambda b, t, sid: (0, 0)),
            pl.BlockSpec((1, DIN, RANK), lambda b, t, sid: (sid[b], 0, 0)),
            pl.BlockSpec((1, RANK, DFF), lambda b, t, sid: (sid[b], 0, 0)),
            pl.BlockSpec((DFF, DIN), lambda b, t, sid: (0, 0)),
            pl.BlockSpec((1, DIN), lambda b, t, sid: (0, 0)),
            pl.BlockSpec((1, DFF, RANK), lambda b, t, sid: (sid[b], 0, 0)),
            pl.BlockSpec((1, RANK, DIN), lambda b, t, sid: (sid[b], 0, 0)),
        ],
        out_specs=pl.BlockSpec((1, TS, DIN), lambda b, t, sid: (b, t, 0)),
    )
    out = pl.pallas_call(
        _fused,
        grid_spec=grid_spec,
        out_shape=jax.ShapeDtypeStruct((B, S, DIN), jnp.float32),
        compiler_params=pltpu.CompilerParams(
            dimension_semantics=("arbitrary", "arbitrary"),
        ),
    )(sid, xb, W1b, b1r, A1b, B1b, W2b, b2r, A2b, B2b)
    return out


# speedup vs baseline: 1.0834x; 1.0834x over previous
import functools

import jax
import jax.numpy as jnp
from jax.experimental import pallas as pl
from jax.experimental.pallas import tpu as pltpu

RANK = 4
ALPHA = 1.0
NSUB = 16
DIN = 1024
DFF = 4096
TS = 1024


def _fused(sid_ref, x_ref, W1_ref, b1_ref, A1_ref, B1_ref,
           W2_ref, b2_ref, A2_ref, B2_ref, out_ref):
    x = x_ref[0]
    scale = ALPHA / RANK
    base = jnp.dot(x, W1_ref[...], preferred_element_type=jnp.float32)
    lo = jnp.dot(x, A1_ref[0], preferred_element_type=jnp.float32)
    lo = jnp.dot(lo.astype(jnp.bfloat16), B1_ref[0],
                 preferred_element_type=jnp.float32)
    h = base + b1_ref[...] + scale * lo
    h = 0.5 * h * (1.0 + jax.lax.erf(h * 0.7071067811865476))
    hb = h.astype(jnp.bfloat16)
    base2 = jnp.dot(hb, W2_ref[...], preferred_element_type=jnp.float32)
    lo2 = jnp.dot(hb, A2_ref[0], preferred_element_type=jnp.float32)
    lo2 = jnp.dot(lo2.astype(jnp.bfloat16), B2_ref[0],
                  preferred_element_type=jnp.float32)
    out_ref[0] = base2 + b2_ref[...] + scale * lo2


def kernel(x, subject_id, W1, b1, A1, B1, W2, b2, A2, B2):
    B, S, _ = x.shape
    bf = jnp.bfloat16
    xb = x.astype(bf)
    W1b, W2b = W1.astype(bf), W2.astype(bf)
    A1b, B1b = A1.astype(bf), B1.astype(bf)
    A2b, B2b = A2.astype(bf), B2.astype(bf)
    b1r = b1.reshape(1, DFF)
    b2r = b2.reshape(1, DIN)
    sid = subject_id.astype(jnp.int32)

    grid = (B, S // TS)
    grid_spec = pltpu.PrefetchScalarGridSpec(
        num_scalar_prefetch=1,
        grid=grid,
        in_specs=[
            pl.BlockSpec((1, TS, DIN), lambda b, t, sid: (b, t, 0)),
            pl.BlockSpec((DIN, DFF), lambda b, t, sid: (0, 0)),
            pl.BlockSpec((1, DFF), lambda b, t, sid: (0, 0)),
            pl.BlockSpec((1, DIN, RANK), lambda b, t, sid: (sid[b], 0, 0)),
            pl.BlockSpec((1, RANK, DFF), lambda b, t, sid: (sid[b], 0, 0)),
            pl.BlockSpec((DFF, DIN), lambda b, t, sid: (0, 0)),
            pl.BlockSpec((1, DIN), lambda b, t, sid: (0, 0)),
            pl.BlockSpec((1, DFF, RANK), lambda b, t, sid: (sid[b], 0, 0)),
            pl.BlockSpec((1, RANK, DIN), lambda b, t, sid: (sid[b], 0, 0)),
        ],
        out_specs=pl.BlockSpec((1, TS, DIN), lambda b, t, sid: (b, t, 0)),
    )
    out = pl.pallas_call(
        _fused,
        grid_spec=grid_spec,
        out_shape=jax.ShapeDtypeStruct((B, S, DIN), jnp.float32),
        compiler_params=pltpu.CompilerParams(
            dimension_semantics=("arbitrary", "arbitrary"),
        ),
    )(sid, xb, W1b, b1r, A1b, B1b, W2b, b2r, A2b, B2b)
    return out


# speedup vs baseline: 1.1437x; 1.0557x over previous
import functools

import jax
import jax.numpy as jnp
from jax.experimental import pallas as pl
from jax.experimental.pallas import tpu as pltpu

RANK = 4
ALPHA = 1.0
NSUB = 16
DIN = 1024
DFF = 4096
TS = 1024


def _fused(sid_ref, x_ref, W1_ref, b1_ref, A1_ref, B1_ref,
           W2_ref, b2_ref, A2_ref, B2_ref, out_ref):
    x = x_ref[0]
    scale = ALPHA / RANK
    base = jnp.dot(x, W1_ref[...], preferred_element_type=jnp.float32)
    lo = scale * jnp.dot(x, A1_ref[0], preferred_element_type=jnp.float32)
    h = base + b1_ref[...]
    B1s = B1_ref[0].astype(jnp.float32)
    for r in range(RANK):
        h = h + lo[:, r:r + 1] * B1s[r:r + 1, :]
    h = 0.5 * h * (1.0 + jax.lax.erf(h * 0.7071067811865476))
    hb = h.astype(jnp.bfloat16)
    base2 = jnp.dot(hb, W2_ref[...], preferred_element_type=jnp.float32)
    lo2 = scale * jnp.dot(hb, A2_ref[0], preferred_element_type=jnp.float32)
    out = base2 + b2_ref[...]
    B2s = B2_ref[0].astype(jnp.float32)
    for r in range(RANK):
        out = out + lo2[:, r:r + 1] * B2s[r:r + 1, :]
    out_ref[0] = out


def kernel(x, subject_id, W1, b1, A1, B1, W2, b2, A2, B2):
    B, S, _ = x.shape
    bf = jnp.bfloat16
    xb = x.astype(bf)
    W1b, W2b = W1.astype(bf), W2.astype(bf)
    A1b, B1b = A1.astype(bf), B1.astype(bf)
    A2b, B2b = A2.astype(bf), B2.astype(bf)
    b1r = b1.reshape(1, DFF)
    b2r = b2.reshape(1, DIN)
    sid = subject_id.astype(jnp.int32)

    grid = (B, S // TS)
    grid_spec = pltpu.PrefetchScalarGridSpec(
        num_scalar_prefetch=1,
        grid=grid,
        in_specs=[
            pl.BlockSpec((1, TS, DIN), lambda b, t, sid: (b, t, 0)),
            pl.BlockSpec((DIN, DFF), lambda b, t, sid: (0, 0)),
            pl.BlockSpec((1, DFF), lambda b, t, sid: (0, 0)),
            pl.BlockSpec((1, DIN, RANK), lambda b, t, sid: (sid[b], 0, 0)),
            pl.BlockSpec((1, RANK, DFF), lambda b, t, sid: (sid[b], 0, 0)),
            pl.BlockSpec((DFF, DIN), lambda b, t, sid: (0, 0)),
            pl.BlockSpec((1, DIN), lambda b, t, sid: (0, 0)),
            pl.BlockSpec((1, DFF, RANK), lambda b, t, sid: (sid[b], 0, 0)),
            pl.BlockSpec((1, RANK, DIN), lambda b, t, sid: (sid[b], 0, 0)),
        ],
        out_specs=pl.BlockSpec((1, TS, DIN), lambda b, t, sid: (b, t, 0)),
    )
    out = pl.pallas_call(
        _fused,
        grid_spec=grid_spec,
        out_shape=jax.ShapeDtypeStruct((B, S, DIN), jnp.float32),
        compiler_params=pltpu.CompilerParams(
            dimension_semantics=("arbitrary", "arbitrary"),
        ),
    )(sid, xb, W1b, b1r, A1b, B1b, W2b, b2r, A2b, B2b)
    return out


# speedup vs baseline: 1.2315x; 1.0768x over previous
import jax
import jax.numpy as jnp
from jax.experimental import pallas as pl
from jax.experimental.pallas import tpu as pltpu

RANK = 4
ALPHA = 1.0
NSUB = 16
DIN = 1024
DFF = 4096
TS = 512


def _fold(sid_ref, W1_ref, A1_ref, B1_ref, W2_ref, A2_ref, B2_ref,
          W1e_ref, W2e_ref):
    scale = ALPHA / RANK
    d1 = jnp.dot(A1_ref[0], B1_ref[0], preferred_element_type=jnp.float32)
    W1e_ref[0] = (W1_ref[...].astype(jnp.float32)
                  + scale * d1).astype(jnp.bfloat16)
    d2 = jnp.dot(A2_ref[0], B2_ref[0], preferred_element_type=jnp.float32)
    W2e_ref[0] = (W2_ref[...].astype(jnp.float32)
                  + scale * d2).astype(jnp.bfloat16)


def _mlp(x_ref, W1e_ref, b1_ref, W2e_ref, b2_ref, out_ref):
    x = x_ref[0]
    h = jnp.dot(x, W1e_ref[0], preferred_element_type=jnp.float32)
    h = h + b1_ref[...]
    h = 0.5 * h * (1.0 + jax.lax.erf(h * 0.7071067811865476))
    out = jnp.dot(h.astype(jnp.bfloat16), W2e_ref[0],
                  preferred_element_type=jnp.float32)
    out_ref[0] = out + b2_ref[...]


def kernel(x, subject_id, W1, b1, A1, B1, W2, b2, A2, B2):
    B, S, _ = x.shape
    bf = jnp.bfloat16
    xb = x.astype(bf)
    A1b, B1b = A1.astype(bf), B1.astype(bf)
    A2b, B2b = A2.astype(bf), B2.astype(bf)
    b1r = b1.reshape(1, DFF)
    b2r = b2.reshape(1, DIN)
    sid = subject_id.astype(jnp.int32)

    fold_spec = pltpu.PrefetchScalarGridSpec(
        num_scalar_prefetch=1,
        grid=(B,),
        in_specs=[
            pl.BlockSpec((DIN, DFF), lambda b, sid: (0, 0)),
            pl.BlockSpec((1, DIN, RANK), lambda b, sid: (sid[b], 0, 0)),
            pl.BlockSpec((1, RANK, DFF), lambda b, sid: (sid[b], 0, 0)),
            pl.BlockSpec((DFF, DIN), lambda b, sid: (0, 0)),
            pl.BlockSpec((1, DFF, RANK), lambda b, sid: (sid[b], 0, 0)),
            pl.BlockSpec((1, RANK, DIN), lambda b, sid: (sid[b], 0, 0)),
        ],
        out_specs=[
            pl.BlockSpec((1, DIN, DFF), lambda b, sid: (b, 0, 0)),
            pl.BlockSpec((1, DFF, DIN), lambda b, sid: (b, 0, 0)),
        ],
    )
    W1e, W2e = pl.pallas_call(
        _fold,
        grid_spec=fold_spec,
        out_shape=[
            jax.ShapeDtypeStruct((B, DIN, DFF), bf),
            jax.ShapeDtypeStruct((B, DFF, DIN), bf),
        ],
        compiler_params=pltpu.CompilerParams(
            dimension_semantics=("arbitrary",),
        ),
    )(sid, W1.astype(bf), A1b, B1b, W2.astype(bf), A2b, B2b)

    out = pl.pallas_call(
        _mlp,
        grid=(B, S // TS),
        in_specs=[
            pl.BlockSpec((1, TS, DIN), lambda b, t: (b, t, 0)),
            pl.BlockSpec((1, DIN, DFF), lambda b, t: (b, 0, 0)),
            pl.BlockSpec((1, DFF), lambda b, t: (0, 0)),
            pl.BlockSpec((1, DFF, DIN), lambda b, t: (b, 0, 0)),
            pl.BlockSpec((1, DIN), lambda b, t: (0, 0)),
        ],
        out_specs=pl.BlockSpec((1, TS, DIN), lambda b, t: (b, t, 0)),
        out_shape=jax.ShapeDtypeStruct((B, S, DIN), jnp.float32),
        compiler_params=pltpu.CompilerParams(
            dimension_semantics=("arbitrary", "arbitrary"),
        ),
    )(xb, W1e, b1r, W2e, b2r)
    return out
